# Initial kernel scaffold; baseline (speedup 1.0000x reference)
import jax
import jax.numpy as jnp
from jax.experimental import pallas as pl


def kernel(z, z_dic):
    raise NotImplementedError("write your pallas kernel here")


# speedup vs baseline: 3.2304x; 3.2304x over previous
import functools

import jax
import jax.numpy as jnp
from jax.experimental import pallas as pl


def _vq_kernel(z_ref, dic_ref, zq_ref, idx_ref):
    dic = dic_ref[...]
    c_norm = jnp.sum(dic * dic, axis=0)
    B = z_ref.shape[0]
    K = dic.shape[1]
    for b in range(B):
        x = z_ref[b]
        dots = jax.lax.dot_general(
            dic, x, (((0,), (0,)), ((), ())),
            preferred_element_type=jnp.float32,
            precision=jax.lax.Precision.HIGHEST)
        dist = c_norm[:, None] - 2.0 * dots
        idx = jnp.argmin(dist, axis=0).astype(jnp.int32)
        idx_ref[b, 0, :] = idx
        onehot = (jax.lax.broadcasted_iota(jnp.int32, (K, x.shape[1]), 0)
                  == idx[None, :]).astype(jnp.float32)
        zq_ref[b] = jax.lax.dot_general(
            dic, onehot, (((1,), (0,)), ((), ())),
            preferred_element_type=jnp.float32,
            precision=jax.lax.Precision.HIGHEST)


@functools.partial(jax.jit, static_argnames=())
def kernel(z, z_dic):
    B, F, H, W = z.shape
    _F, K = z_dic.shape
    HW = H * W
    z_r = z.reshape(B, F, HW)
    zq_r, idx_r = pl.pallas_call(
        _vq_kernel,
        out_shape=(
            jax.ShapeDtypeStruct((B, F, HW), jnp.float32),
            jax.ShapeDtypeStruct((B, 1, HW), jnp.int32),
        ),
    )(z_r, z_dic)
    return (zq_r.reshape(B, F, H, W), idx_r.reshape(B, H, W))
